# baseline (speedup 1.0000x reference)
import functools

import jax
import jax.numpy as jnp
from jax import lax
from jax.experimental import pallas as pl
from jax.experimental.pallas import tpu as pltpu
from jax.experimental.pallas import tpu_sc as plsc

_NC = 2
_NW = 32


def _tc_body(row_ref, col_ref, out_ref):
    r = row_ref.shape[0]
    nf = row_ref.shape[2]
    w = col_ref.shape[0]
    col = col_ref[...]
    row = row_ref[...]
    out_ref[:, :, 0:nf] = jnp.broadcast_to(col[None, :, :], (r, w, nf))
    out_ref[:, :, nf : 2 * nf] = jnp.broadcast_to(row, (r, w, nf))


def _tc_part(row_part, col_embed, r):
    hp, nf = row_part.shape
    w = col_embed.shape[0]
    return pl.pallas_call(
        _tc_body,
        grid=(hp // r,),
        in_specs=[
            pl.BlockSpec((r, 1, nf), lambda i: (i, 0, 0)),
            pl.BlockSpec((w, nf), lambda i: (0, 0)),
        ],
        out_specs=pl.BlockSpec((r, w, 2 * nf), lambda i: (i, 0, 0)),
        out_shape=jax.ShapeDtypeStruct((hp, w, 2 * nf), jnp.float32),
    )(row_part.reshape(hp, 1, nf), col_embed)


def _sc_pos_kernel(h, w, nf, row_hbm, col_hbm, out_hbm, buf0, buf1, row_v, sem0, sem1):
    wid = lax.axis_index("s") * _NC + lax.axis_index("c")

    pltpu.sync_copy(col_hbm, buf0.at[:, pl.ds(0, nf)])
    pltpu.sync_copy(col_hbm, buf1.at[:, pl.ds(0, nf)])

    n_iter = (h + _NW - 1) // _NW
    bufs = (buf0, buf1)
    sems = (sem0, sem1)
    nreg = nf // 16

    for t in range(n_iter):
        i = wid + _NW * t
        buf = bufs[t % 2]
        sem = sems[t % 2]

        @pl.when(i < h)
        def _():
            if t >= 2:
                pltpu.make_async_copy(buf, out_hbm.at[i], sem).wait()
            pltpu.sync_copy(row_hbm.at[pl.ds(i, 1), :], row_v)
            regs = [row_v.at[pl.ds(0, 1), pl.ds(16 * c, 16)][...] for c in range(nreg)]

            @pl.loop(0, w)
            def _(r):
                for c in range(nreg):
                    buf.at[pl.ds(r, 1), pl.ds(nf + 16 * c, 16)][...] = regs[c]

            pltpu.async_copy(buf, out_hbm.at[i], sem)

    for t in range(n_iter):
        i = wid + _NW * t

        @pl.when((i < h) & (i + 2 * _NW >= h))
        def _():
            pltpu.make_async_copy(bufs[t % 2], out_hbm.at[i], sems[t % 2]).wait()


def _sc_part(row_part, col_embed):
    hp, nf = row_part.shape
    w = col_embed.shape[0]
    mesh = plsc.VectorSubcoreMesh(core_axis_name="c", subcore_axis_name="s")
    k = pl.kernel(
        functools.partial(_sc_pos_kernel, hp, w, nf),
        out_type=jax.ShapeDtypeStruct((hp, w, 2 * nf), jnp.float32),
        mesh=mesh,
        scratch_types=[
            pltpu.VMEM((w, 2 * nf), jnp.float32),
            pltpu.VMEM((w, 2 * nf), jnp.float32),
            pltpu.VMEM((1, nf), jnp.float32),
            pltpu.SemaphoreType.DMA,
            pltpu.SemaphoreType.DMA,
        ],
    )
    return k(row_part, col_embed)


def kernel(row_embed, col_embed, bev_h, bev_w):
    h, nf = row_embed.shape
    w, _ = col_embed.shape
    h_tc = 160
    tc = _tc_part(row_embed[:h_tc], col_embed, r=20)
    sc = _sc_part(row_embed[h_tc:], col_embed)
    out = jnp.concatenate([tc, sc], axis=0)
    return out.reshape(1, h * w, 2 * nf)


# speedup vs baseline: 4.7958x; 4.7958x over previous
import jax
import jax.numpy as jnp
from jax.experimental import pallas as pl
from jax.experimental.pallas import tpu as pltpu

_R = 20
_NBUF = 4


def _pos_body(row_ref, col_ref, out_ref, *rest):
    h = row_ref.shape[0]
    nf = row_ref.shape[2]
    w = col_ref.shape[0]
    n_blk = h // _R
    bufs = rest[:_NBUF]
    sems = rest[_NBUF:]

    left = jnp.broadcast_to(col_ref[...][None, :, :], (_R, w, nf))
    for b in range(_NBUF):
        bufs[b][:, :, 0:nf] = left

    for t in range(n_blk):
        b = t % _NBUF
        if t >= _NBUF:
            pltpu.make_async_copy(
                bufs[b], out_ref.at[pl.ds((t - _NBUF) * _R, _R)], sems[b]
            ).wait()
        rowv = row_ref[pl.ds(t * _R, _R)]
        bufs[b][:, :, nf : 2 * nf] = jnp.broadcast_to(rowv, (_R, w, nf))
        pltpu.async_copy(bufs[b], out_ref.at[pl.ds(t * _R, _R)], sems[b])

    for t in range(max(0, n_blk - _NBUF), n_blk):
        b = t % _NBUF
        pltpu.make_async_copy(bufs[b], out_ref.at[pl.ds(t * _R, _R)], sems[b]).wait()


def kernel(row_embed, col_embed, bev_h, bev_w):
    h, nf = row_embed.shape
    w, _ = col_embed.shape
    out = pl.pallas_call(
        _pos_body,
        in_specs=[
            pl.BlockSpec(memory_space=pltpu.VMEM),
            pl.BlockSpec(memory_space=pltpu.VMEM),
        ],
        out_specs=pl.BlockSpec(memory_space=pl.ANY),
        out_shape=jax.ShapeDtypeStruct((h, w, 2 * nf), jnp.float32),
        scratch_shapes=(
            [pltpu.VMEM((_R, w, 2 * nf), jnp.float32) for _ in range(_NBUF)]
            + [pltpu.SemaphoreType.DMA for _ in range(_NBUF)]
        ),
    )(row_embed.reshape(h, 1, nf), col_embed)
    return out.reshape(1, h * w, 2 * nf)
